# Initial kernel scaffold; baseline (speedup 1.0000x reference)
import jax
import jax.numpy as jnp
from jax.experimental import pallas as pl


def kernel(src, time, weekday, duration, emb_loc, minute_table, hour_table, weekday_table, duration_table):
    raise NotImplementedError("write your pallas kernel here")


# speedup vs baseline: 8.3765x; 8.3765x over previous
import math

import jax
import jax.numpy as jnp
import numpy as np
from jax import lax
from jax.experimental import pallas as pl
from jax.experimental.pallas import tpu as pltpu
from jax.experimental.pallas import tpu_sc as plsc

D = 64
S = 200
B = 1024
N = S * B
C = 256
NC = 2
NS = 16
NW = NC * NS
CHUNKS = N // C
PER_W = CHUNKS // NW
CPS = B // C
HALF = 128

_SQRT_D = float(math.sqrt(D))


def _pe_table():
    den = np.exp(-np.arange(0, D, 2) * math.log(10000.0) / D)
    pos = np.arange(0, S).reshape(S, 1)
    pe = np.zeros((S, D), dtype=np.float32)
    pe[:, 0::2] = np.sin(pos * den)
    pe[:, 1::2] = np.cos(pos * den)
    return jnp.asarray(pe)


def _sc_body(src_h, time_h, wd_h, dur_h, emb_h, comb_h, durt_h, pe_h, out_h,
             sidx, tbuf, wbuf, cidx, didx, rows, crows, drows, pe_v, sem):
    wid = lax.axis_index("s") * NC + lax.axis_index("c")

    def chunk_body(k, carry):
        chunk = wid * PER_W + k
        base = chunk * C
        s = chunk // CPS

        small = []
        for j in range(C // HALF):
            sl = pl.ds(base + j * HALF, HALF)
            small.append(pltpu.async_copy(src_h.at[sl], sidx.at[j], sem))
            small.append(pltpu.async_copy(time_h.at[sl], tbuf.at[j], sem))
            small.append(pltpu.async_copy(wd_h.at[sl], wbuf.at[j], sem))
            small.append(pltpu.async_copy(dur_h.at[sl], didx.at[j], sem))
        small.append(pltpu.async_copy(pe_h.at[s], pe_v, sem))
        for cp in small:
            cp.wait()

        def cpass(i, c):
            for j in range(C // HALF):
                t = tbuf[j, pl.ds(i * 16, 16)]
                w = wbuf[j, pl.ds(i * 16, 16)]
                cidx[j, pl.ds(i * 16, 16)] = t * 7 + w
            return c
        lax.fori_loop(0, HALF // 16, cpass, 0)

        gs = []
        for j in range(C // HALF):
            dst = pl.ds(j * HALF, HALF)
            gs.append(pltpu.async_copy(emb_h.at[sidx.at[j]], rows.at[dst], sem))
            gs.append(pltpu.async_copy(comb_h.at[cidx.at[j]], crows.at[dst], sem))
            gs.append(pltpu.async_copy(durt_h.at[didx.at[j]], drows.at[dst], sem))
        for cp in gs:
            cp.wait()

        pe_regs = [pe_v[pl.ds(d * 16, 16)] for d in range(D // 16)]

        def tok(t, c):
            for d in range(D // 16):
                sl = pl.ds(d * 16, 16)
                acc = (rows[t, sl] + crows[t, sl] + drows[t, sl])
                rows[t, sl] = acc * _SQRT_D + pe_regs[d]
            return c
        lax.fori_loop(0, C, tok, 0)

        pltpu.sync_copy(rows, out_h.at[pl.ds(base, C)])
        return carry

    lax.fori_loop(0, PER_W, chunk_body, 0)


def kernel(src, time, weekday, duration, emb_loc, minute_table, hour_table,
           weekday_table, duration_table):
    src_i = src.reshape(N).astype(jnp.int32)
    time_i = time.reshape(N).astype(jnp.int32)
    wd_i = weekday.reshape(N).astype(jnp.int32)
    dur_i = duration.reshape(N).astype(jnp.int32)

    tw = (hour_table[:24, None, :] + minute_table[None, :4, :]).reshape(96, D)
    comb = (tw[:, None, :] + weekday_table[None, :7, :]).reshape(96 * 7, D)
    pe = _pe_table()

    mesh = plsc.VectorSubcoreMesh(core_axis_name="c", subcore_axis_name="s",
                                  num_cores=NC, num_subcores=NS)
    k = pl.kernel(
        _sc_body,
        out_type=jax.ShapeDtypeStruct((N, D), jnp.float32),
        mesh=mesh,
        compiler_params=pltpu.CompilerParams(use_tc_tiling_on_sc=False),
        scratch_types=[
            pltpu.VMEM((C // HALF, HALF), jnp.int32),
            pltpu.VMEM((C // HALF, HALF), jnp.int32),
            pltpu.VMEM((C // HALF, HALF), jnp.int32),
            pltpu.VMEM((C // HALF, HALF), jnp.int32),
            pltpu.VMEM((C // HALF, HALF), jnp.int32),
            pltpu.VMEM((C, D), jnp.float32),
            pltpu.VMEM((C, D), jnp.float32),
            pltpu.VMEM((C, D), jnp.float32),
            pltpu.VMEM((D,), jnp.float32),
            pltpu.SemaphoreType.DMA,
        ],
    )
    out = k(src_i, time_i, wd_i, dur_i, emb_loc, comb, duration_table, pe)
    return out.reshape(S, B, D)
